# baseline (speedup 1.0000x reference)
import functools

import jax
import jax.numpy as jnp
from jax import lax
from jax.experimental import pallas as pl
from jax.experimental.pallas import tpu as pltpu
from jax.experimental.pallas import tpu_sc as plsc


def _make_sc_kernel(m, s, i, slots, h, dtype):
    info = plsc.get_sparse_core_info()
    nw = info.num_cores * info.num_subcores
    b_per_w = m // nw
    mesh = plsc.VectorSubcoreMesh(core_axis_name="c", subcore_axis_name="s")

    @functools.partial(
        pl.kernel,
        mesh=mesh,
        out_type=(
            jax.ShapeDtypeStruct((m, s, i), dtype),
            jax.ShapeDtypeStruct((m, h), dtype),
        ),
        scratch_types=[
            pltpu.VMEM((b_per_w,), jnp.int32),
            pltpu.VMEM((b_per_w, h), dtype),
            pltpu.VMEM((b_per_w, h), dtype),
            pltpu.SemaphoreType.DMA,
            pltpu.SemaphoreType.DMA,
            pltpu.SemaphoreType.DMA,
        ],
    )
    def sc_k(top_hbm, trees_hbm, shifted_hbm, hid_hbm, newt_hbm, head_hbm,
             top_v, rows_v, shifted_v, sem_bulk, sem_g, sem_s):
        wid = lax.axis_index("s") * info.num_cores + lax.axis_index("c")
        base = wid * b_per_w
        slab = pl.ds(base, b_per_w)
        bulk = pltpu.make_async_copy(trees_hbm.at[slab], newt_hbm.at[slab], sem_bulk)
        bulk.start()
        pltpu.sync_copy(top_hbm.at[slab], top_v)
        pltpu.sync_copy(shifted_hbm.at[slab], shifted_v)
        for c in range(b_per_w // 16):
            tv = top_v[pl.ds(c * 16, 16)]
            for k in range(16):
                j = c * 16 + k
                t = tv[k]
                pltpu.make_async_copy(
                    hid_hbm.at[base + j, pl.ds(t + 1, 1)],
                    rows_v.at[pl.ds(j, 1)],
                    sem_g,
                ).start()
        bulk.wait()
        for c in range(b_per_w // 16):
            tv = top_v[pl.ds(c * 16, 16)]
            for k in range(16):
                j = c * 16 + k
                t = tv[k]
                pltpu.make_async_copy(
                    shifted_v.at[pl.ds(j, 1)],
                    newt_hbm.at[base + j, pl.ds(t, 1)],
                    sem_s,
                ).start()
        pltpu.make_async_copy(head_hbm.at[slab], rows_v, sem_g).wait()
        pltpu.sync_copy(rows_v, head_hbm.at[slab])
        pltpu.make_async_copy(head_hbm.at[slab], rows_v, sem_s).wait()

    return sc_k


def kernel(trees, hiddens, shifted_embs, top_position):
    m, s, i = trees.shape
    slots = hiddens.shape[1]
    h = hiddens.shape[2]
    new_trees, hidden_head = _make_sc_kernel(m, s, i, slots, h, trees.dtype)(
        top_position, trees, shifted_embs, hiddens
    )
    return (new_trees, hidden_head)


# speedup vs baseline: 18.4906x; 18.4906x over previous
import functools

import jax
import jax.numpy as jnp
from jax import lax
from jax.experimental import pallas as pl
from jax.experimental.pallas import tpu as pltpu
from jax.experimental.pallas import tpu_sc as plsc

ROWS_PER_BLOCK = 256


def _trees_body(top_ref, trees_ref, shifted_ref, out_ref):
    r, s, i = trees_ref.shape
    stack_iota = lax.broadcasted_iota(jnp.int32, (r, s, i), 1)
    mask = stack_iota == top_ref[...]
    out_ref[...] = jnp.where(mask, shifted_ref[...], trees_ref[...])


def _make_trees_call(m, s, i, dtype):
    r = ROWS_PER_BLOCK
    grid = (m // r,)
    return pl.pallas_call(
        _trees_body,
        grid=grid,
        in_specs=[
            pl.BlockSpec((r, 1, 1), lambda g: (g, 0, 0)),
            pl.BlockSpec((r, s, i), lambda g: (g, 0, 0)),
            pl.BlockSpec((r, 1, i), lambda g: (g, 0, 0)),
        ],
        out_specs=pl.BlockSpec((r, s, i), lambda g: (g, 0, 0)),
        out_shape=jax.ShapeDtypeStruct((m, s, i), dtype),
    )


def _make_hidden_gather(m, slots, h, dtype):
    info = plsc.get_sparse_core_info()
    nw = info.num_cores * info.num_subcores
    b_per_w = m // nw
    mesh = plsc.VectorSubcoreMesh(core_axis_name="c", subcore_axis_name="s")

    @functools.partial(
        pl.kernel,
        mesh=mesh,
        out_type=jax.ShapeDtypeStruct((m, h), dtype),
        scratch_types=[
            pltpu.VMEM((b_per_w,), jnp.int32),
            pltpu.VMEM((b_per_w, h), dtype),
            pltpu.SemaphoreType.DMA,
        ],
    )
    def gather_k(top_hbm, hid_hbm, out_hbm, top_v, rows_v, sem):
        wid = lax.axis_index("s") * info.num_cores + lax.axis_index("c")
        base = wid * b_per_w
        pltpu.sync_copy(top_hbm.at[pl.ds(base, b_per_w)], top_v)
        for c in range(b_per_w // 16):
            tv = top_v[pl.ds(c * 16, 16)]
            for k in range(16):
                j = c * 16 + k
                t = tv[k] + 1
                pltpu.make_async_copy(
                    hid_hbm.at[base + j, pl.ds(t, 1)], rows_v.at[pl.ds(j, 1)], sem
                ).start()
        pltpu.make_async_copy(
            out_hbm.at[pl.ds(base, b_per_w)], rows_v, sem
        ).wait()
        pltpu.sync_copy(rows_v, out_hbm.at[pl.ds(base, b_per_w)])

    return gather_k


def kernel(trees, hiddens, shifted_embs, top_position):
    m, s, i = trees.shape
    slots = hiddens.shape[1]
    h = hiddens.shape[2]
    new_trees = _make_trees_call(m, s, i, trees.dtype)(
        top_position.reshape(m, 1, 1), trees, shifted_embs.reshape(m, 1, i)
    )
    hidden_head = _make_hidden_gather(m, slots, h, hiddens.dtype)(
        top_position, hiddens
    )
    return (new_trees, hidden_head)
